# baseline (speedup 1.0000x reference)
import jax
import jax.numpy as jnp
from jax import lax
from jax.experimental import pallas as pl
from jax.experimental.pallas import tpu as pltpu
from jax.experimental.pallas import tpu_sc as plsc

N = 10000
E = 320000
D = 128
N_CLASSES = 10
N_GRAPHS = 64

NC = 2
NS = 16
NW = NC * NS

CHUNK = 128
NCHUNKS = 80
EDGES_PER_TILE = CHUNK * NCHUNKS
E_PAD = EDGES_PER_TILE * NW
N_PAD = 10240
ROWS_PER_TILE = N_PAD // NS
DEGW = 16

ROW_BLK = 1000
GRID = N // ROW_BLK

_f32 = jnp.float32


ZROWS = 16


def _zero_vmem_block(zbuf, ncols):
    def zrow(r, carry):
        for cc in range(ncols // 16):
            zbuf[r, pl.ds(cc * 16, 16)] = jnp.zeros((16,), _f32)
        return carry
    lax.fori_loop(0, ZROWS, zrow, 0)


def _zero_spmem_rows(acc, zbuf, row0, nrows):
    def zchunk(j, carry):
        pltpu.sync_copy(zbuf, acc.at[pl.ds(row0 + j * ZROWS, ZROWS)])
        return carry
    lax.fori_loop(0, nrows // ZROWS, zchunk, 0)


def _deg_body(dst_hbm, out_hbm, acc, zbuf, ones_v, idx_d):
    c = lax.axis_index("c")
    s = lax.axis_index("s")
    wid = c * NS + s
    row0 = s * ROWS_PER_TILE
    _zero_vmem_block(zbuf, DEGW)
    _zero_spmem_rows(acc, zbuf, row0, ROWS_PER_TILE)

    def orow(r, carry):
        ones_v[r] = jnp.ones((16,), _f32)
        return carry
    lax.fori_loop(0, CHUNK, orow, 0)
    plsc.subcore_barrier()

    ebase = wid * EDGES_PER_TILE

    def step(i, carry):
        pltpu.sync_copy(dst_hbm.at[pl.ds(ebase + i * CHUNK, CHUNK)], idx_d)
        pltpu.sync_copy(ones_v, acc.at[idx_d], add=True)
        return carry
    lax.fori_loop(0, NCHUNKS, step, 0)
    plsc.subcore_barrier()
    pltpu.sync_copy(acc.at[pl.ds(row0, ROWS_PER_TILE)],
                    out_hbm.at[c, pl.ds(row0, ROWS_PER_TILE)])


def _conv_body(src_hbm, dst_hbm, g_hbm, out_hbm, acc, zbuf, idx_s,
               dbuf0, dbuf1, rows0, rows1, semg0, semg1, semd0, semd1):
    c = lax.axis_index("c")
    s = lax.axis_index("s")
    wid = c * NS + s
    row0 = s * ROWS_PER_TILE
    _zero_vmem_block(zbuf, D)
    _zero_spmem_rows(acc, zbuf, row0, ROWS_PER_TILE)
    pltpu.sync_copy(src_hbm.at[wid], idx_s)
    plsc.subcore_barrier()

    ebase = wid * EDGES_PER_TILE

    pltpu.async_copy(dst_hbm.at[pl.ds(ebase, CHUNK)], dbuf0, semd0)
    pltpu.async_copy(dst_hbm.at[pl.ds(ebase + CHUNK, CHUNK)], dbuf1, semd1)
    pltpu.async_copy(g_hbm.at[idx_s.at[0]], rows0, semg0)
    pltpu.async_copy(g_hbm.at[idx_s.at[1]], rows1, semg1)

    def half(i, dbuf, rows, semd, semg):
        pltpu.make_async_copy(g_hbm.at[idx_s.at[i]], rows, semg).wait()
        pltpu.make_async_copy(dst_hbm.at[pl.ds(ebase, CHUNK)], dbuf,
                              semd).wait()
        pltpu.sync_copy(rows, acc.at[dbuf], add=True)

        @pl.when(i < NCHUNKS - 2)
        def _():
            pltpu.async_copy(
                dst_hbm.at[pl.ds(ebase + (i + 2) * CHUNK, CHUNK)], dbuf, semd)
            pltpu.async_copy(g_hbm.at[idx_s.at[i + 2]], rows, semg)

    def body(j, carry):
        half(2 * j, dbuf0, rows0, semd0, semg0)
        half(2 * j + 1, dbuf1, rows1, semd1, semg1)
        return carry
    lax.fori_loop(0, NCHUNKS // 2, body, 0)
    plsc.subcore_barrier()
    pltpu.sync_copy(acc.at[pl.ds(row0, ROWS_PER_TILE)],
                    out_hbm.at[c, pl.ds(row0, ROWS_PER_TILE)])


_sc_kernels_cache = {}


def _get_sc_kernels():
    if "k" not in _sc_kernels_cache:
        mesh = plsc.VectorSubcoreMesh(core_axis_name="c", subcore_axis_name="s")
        deg_k = pl.kernel(
            _deg_body,
            out_type=jax.ShapeDtypeStruct((NC, N_PAD, DEGW), _f32),
            mesh=mesh,
            scratch_types=[
                pltpu.VMEM_SHARED((N_PAD, DEGW), _f32),
                pltpu.VMEM((ZROWS, DEGW), _f32),
                pltpu.VMEM((CHUNK, DEGW), _f32),
                pltpu.VMEM((CHUNK,), jnp.int32),
            ],
        )
        conv_k = pl.kernel(
            _conv_body,
            out_type=jax.ShapeDtypeStruct((NC, N_PAD, D), _f32),
            mesh=mesh,
            scratch_types=[
                pltpu.VMEM_SHARED((N_PAD, D), _f32),
                pltpu.VMEM((ZROWS, D), _f32),
                pltpu.VMEM((NCHUNKS, CHUNK), jnp.int32),
                pltpu.VMEM((CHUNK,), jnp.int32),
                pltpu.VMEM((CHUNK,), jnp.int32),
                pltpu.VMEM((CHUNK, D), _f32),
                pltpu.VMEM((CHUNK, D), _f32),
                pltpu.SemaphoreType.DMA,
                pltpu.SemaphoreType.DMA,
                pltpu.SemaphoreType.DMA,
                pltpu.SemaphoreType.DMA,
            ],
        )
        _sc_kernels_cache["k"] = (deg_k, conv_k)
    return _sc_kernels_cache["k"]


def _dinv(d0_ref, d1_ref):
    d = d0_ref[:, 0:1] + d1_ref[:, 0:1] + 1.0
    return lax.rsqrt(d)


def _ka_body(x_ref, d0_ref, d1_ref, w_ref, o_ref):
    dinv = _dinv(d0_ref, d1_ref)
    o_ref[...] = jnp.dot(x_ref[...], w_ref[...],
                         preferred_element_type=_f32) * dinv


def _kb_body(a0_ref, a1_ref, g_ref, d0_ref, d1_ref, bias_ref, w_ref, o_ref):
    dinv = _dinv(d0_ref, d1_ref)
    h = (a0_ref[...] + a1_ref[...] + g_ref[...]) * dinv + bias_ref[...]
    h = jnp.maximum(h, 0.0)
    o_ref[...] = jnp.dot(h, w_ref[...], preferred_element_type=_f32) * dinv


def _kc_body(a0_ref, a1_ref, g_ref, d0_ref, d1_ref, bias_ref, bb_ref,
             sums_ref, cnts_ref):
    i = pl.program_id(0)
    dinv = _dinv(d0_ref, d1_ref)
    h = (a0_ref[...] + a1_ref[...] + g_ref[...]) * dinv + bias_ref[...]
    h = jnp.maximum(h, 0.0)
    ids = bb_ref[:, 0:1].astype(jnp.int32)
    iota = lax.broadcasted_iota(jnp.int32, (ROW_BLK, N_GRAPHS), 1)
    oh = jnp.where(ids == iota, 1.0, 0.0)
    dn = (((0,), (0,)), ((), ()))
    ps = lax.dot_general(oh, h, dn, preferred_element_type=_f32)
    pc = lax.dot_general(oh, jnp.ones((ROW_BLK, D), _f32), dn,
                         preferred_element_type=_f32)

    @pl.when(i == 0)
    def _():
        sums_ref[...] = jnp.zeros_like(sums_ref)
        cnts_ref[...] = jnp.zeros_like(cnts_ref)

    sums_ref[...] += ps
    cnts_ref[...] += pc


def _kd_body(s_ref, c_ref, w_ref, bl_ref, o_ref):
    pooled = s_ref[...] / jnp.maximum(c_ref[...], 1.0)
    o_ref[...] = jnp.dot(pooled, w_ref[...],
                         preferred_element_type=_f32) + bl_ref[...]


def _row_spec(cols):
    return pl.BlockSpec((ROW_BLK, cols), lambda i: (i, 0))


def _full_spec(shape):
    return pl.BlockSpec(shape, lambda i: tuple(0 for _ in shape))


_KA = pl.pallas_call(
    _ka_body,
    grid=(GRID,),
    in_specs=[_row_spec(D), _row_spec(DEGW), _row_spec(DEGW),
              _full_spec((D, D))],
    out_specs=_row_spec(D),
    out_shape=jax.ShapeDtypeStruct((N, D), _f32),
)

_KB = pl.pallas_call(
    _kb_body,
    grid=(GRID,),
    in_specs=[_row_spec(D), _row_spec(D), _row_spec(D),
              _row_spec(DEGW), _row_spec(DEGW),
              _full_spec((1, D)), _full_spec((D, D))],
    out_specs=_row_spec(D),
    out_shape=jax.ShapeDtypeStruct((N, D), _f32),
)

_KC = pl.pallas_call(
    _kc_body,
    grid=(GRID,),
    in_specs=[_row_spec(D), _row_spec(D), _row_spec(D),
              _row_spec(DEGW), _row_spec(DEGW),
              _full_spec((1, D)), _row_spec(8)],
    out_specs=[_full_spec((N_GRAPHS, D)), _full_spec((N_GRAPHS, D))],
    out_shape=[jax.ShapeDtypeStruct((N_GRAPHS, D), _f32),
               jax.ShapeDtypeStruct((N_GRAPHS, D), _f32)],
)

_KD = pl.pallas_call(
    _kd_body,
    in_specs=[pl.BlockSpec((N_GRAPHS, D), lambda: (0, 0)),
              pl.BlockSpec((N_GRAPHS, D), lambda: (0, 0)),
              pl.BlockSpec((D, N_CLASSES), lambda: (0, 0)),
              pl.BlockSpec((1, N_CLASSES), lambda: (0, 0))],
    out_specs=pl.BlockSpec((N_GRAPHS, N_CLASSES), lambda: (0, 0)),
    out_shape=jax.ShapeDtypeStruct((N_GRAPHS, N_CLASSES), _f32),
)


def kernel(x, e, b, W1, b1, W2, b2, Wlin, blin):
    src = e[0].astype(jnp.int32)
    dst = e[1].astype(jnp.int32)
    pad = E_PAD - E
    src_p = jnp.concatenate([src, jnp.zeros((pad,), jnp.int32)])
    src_p = src_p.reshape(NW, NCHUNKS, CHUNK)
    dst_f = jnp.concatenate([dst, jnp.full((pad,), N, jnp.int32)])

    deg_k, conv_k = _get_sc_kernels()
    degp = deg_k(dst_f)
    d0 = degp[0, :N]
    d1 = degp[1, :N]

    g1 = _KA(x, d0, d1, W1)
    accA = conv_k(src_p, dst_f, g1)
    g2 = _KB(accA[0, :N], accA[1, :N], g1, d0, d1,
             b1.reshape(1, D), W2)
    accB = conv_k(src_p, dst_f, g2)

    bb = jnp.broadcast_to(b.astype(_f32)[:, None], (N, 8))
    sums, cnts = _KC(accB[0, :N], accB[1, :N], g2, d0, d1,
                     b2.reshape(1, D), bb)
    return _KD(sums, cnts, Wlin, blin.reshape(1, N_CLASSES))


# speedup vs baseline: 3.2024x; 3.2024x over previous
import jax
import jax.numpy as jnp
from jax import lax
from jax.experimental import pallas as pl
from jax.experimental.pallas import tpu as pltpu
from jax.experimental.pallas import tpu_sc as plsc

N = 10000
E = 320000
D = 128
N_CLASSES = 10
N_GRAPHS = 64

NC = 2
NS = 16
NW = NC * NS

CHUNK = 128
NCHUNKS = 80
EDGES_PER_TILE = CHUNK * NCHUNKS
E_PAD = EDGES_PER_TILE * NW
N_PAD = 10240
ROWS_PER_TILE = N_PAD // NS
DEGW = 16

ROW_BLK = 1000
GRID = N // ROW_BLK

_f32 = jnp.float32


ZROWS = 16


def _zero_vmem_block(zbuf, ncols):
    def zrow(r, carry):
        for cc in range(ncols // 16):
            zbuf[r, pl.ds(cc * 16, 16)] = jnp.zeros((16,), _f32)
        return carry
    lax.fori_loop(0, ZROWS, zrow, 0)


def _zero_spmem_rows(acc, zbuf, row0, nrows):
    def zchunk(j, carry):
        pltpu.sync_copy(zbuf, acc.at[pl.ds(row0 + j * ZROWS, ZROWS)])
        return carry
    lax.fori_loop(0, nrows // ZROWS, zchunk, 0)


def _deg_body(dst_hbm, out_hbm, acc, zbuf, ones_v, idx_d):
    c = lax.axis_index("c")
    s = lax.axis_index("s")
    wid = c * NS + s
    row0 = s * ROWS_PER_TILE
    _zero_vmem_block(zbuf, DEGW)
    _zero_spmem_rows(acc, zbuf, row0, ROWS_PER_TILE)

    def orow(r, carry):
        ones_v[r] = jnp.ones((16,), _f32)
        return carry
    lax.fori_loop(0, CHUNK, orow, 0)
    plsc.subcore_barrier()

    ebase = wid * EDGES_PER_TILE

    def step(i, carry):
        pltpu.sync_copy(dst_hbm.at[pl.ds(ebase + i * CHUNK, CHUNK)], idx_d)
        pltpu.sync_copy(ones_v, acc.at[idx_d], add=True)
        return carry
    lax.fori_loop(0, NCHUNKS, step, 0)
    plsc.subcore_barrier()
    pltpu.sync_copy(acc.at[pl.ds(row0, ROWS_PER_TILE)],
                    out_hbm.at[c, pl.ds(row0, ROWS_PER_TILE)])


def _conv_body(src_hbm, dst_hbm, g_hbm, out_hbm, acc, zbuf, idx_s,
               dbuf0, dbuf1, rows0, rows1, semg0, semg1, semd0, semd1):
    c = lax.axis_index("c")
    s = lax.axis_index("s")
    wid = c * NS + s
    row0 = s * ROWS_PER_TILE
    _zero_vmem_block(zbuf, D)
    _zero_spmem_rows(acc, zbuf, row0, ROWS_PER_TILE)
    pltpu.sync_copy(src_hbm.at[wid], idx_s)
    plsc.subcore_barrier()

    ebase = wid * EDGES_PER_TILE

    pltpu.async_copy(dst_hbm.at[pl.ds(ebase, CHUNK)], dbuf0, semd0)
    pltpu.async_copy(dst_hbm.at[pl.ds(ebase + CHUNK, CHUNK)], dbuf1, semd1)
    pltpu.async_copy(g_hbm.at[idx_s.at[0]], rows0, semg0)
    pltpu.async_copy(g_hbm.at[idx_s.at[1]], rows1, semg1)

    def half(i, dbuf, rows, semd, semg):
        pltpu.make_async_copy(g_hbm.at[idx_s.at[i]], rows, semg).wait()
        pltpu.make_async_copy(dst_hbm.at[pl.ds(ebase, CHUNK)], dbuf,
                              semd).wait()
        pltpu.sync_copy(rows, acc.at[dbuf], add=True)

        @pl.when(i < NCHUNKS - 2)
        def _():
            pltpu.async_copy(
                dst_hbm.at[pl.ds(ebase + (i + 2) * CHUNK, CHUNK)], dbuf, semd)
            pltpu.async_copy(g_hbm.at[idx_s.at[i + 2]], rows, semg)

    def body(j, carry):
        half(2 * j, dbuf0, rows0, semd0, semg0)
        half(2 * j + 1, dbuf1, rows1, semd1, semg1)
        return carry
    lax.fori_loop(0, NCHUNKS // 2, body, 0)
    plsc.subcore_barrier()
    pltpu.sync_copy(acc.at[pl.ds(row0, ROWS_PER_TILE)],
                    out_hbm.at[c, pl.ds(row0, ROWS_PER_TILE)])


_sc_kernels_cache = {}


def _get_sc_kernels():
    if "k" not in _sc_kernels_cache:
        mesh = plsc.VectorSubcoreMesh(core_axis_name="c", subcore_axis_name="s")
        deg_k = pl.kernel(
            _deg_body,
            out_type=jax.ShapeDtypeStruct((NC, N_PAD, DEGW), _f32),
            mesh=mesh,
            scratch_types=[
                pltpu.VMEM_SHARED((N_PAD, DEGW), _f32),
                pltpu.VMEM((ZROWS, DEGW), _f32),
                pltpu.VMEM((CHUNK, DEGW), _f32),
                pltpu.VMEM((CHUNK,), jnp.int32),
            ],
        )
        conv_k = pl.kernel(
            _conv_body,
            out_type=jax.ShapeDtypeStruct((NC, N_PAD, D), _f32),
            mesh=mesh,
            scratch_types=[
                pltpu.VMEM_SHARED((N_PAD, D), _f32),
                pltpu.VMEM((ZROWS, D), _f32),
                pltpu.VMEM((NCHUNKS, CHUNK), jnp.int32),
                pltpu.VMEM((CHUNK,), jnp.int32),
                pltpu.VMEM((CHUNK,), jnp.int32),
                pltpu.VMEM((CHUNK, D), _f32),
                pltpu.VMEM((CHUNK, D), _f32),
                pltpu.SemaphoreType.DMA,
                pltpu.SemaphoreType.DMA,
                pltpu.SemaphoreType.DMA,
                pltpu.SemaphoreType.DMA,
            ],
        )
        _sc_kernels_cache["k"] = (deg_k, conv_k)
    return _sc_kernels_cache["k"]


def _dinv(d0_ref, d1_ref):
    d = d0_ref[:, 0:1] + d1_ref[:, 0:1] + 1.0
    return lax.rsqrt(d)


def _ka_body(x_ref, d0_ref, d1_ref, w_ref, o_ref):
    dinv = _dinv(d0_ref, d1_ref)
    o_ref[...] = jnp.dot(x_ref[...], w_ref[...],
                         preferred_element_type=_f32) * dinv


def _kb_body(a0_ref, a1_ref, g_ref, d0_ref, d1_ref, bias_ref, w_ref, o_ref):
    dinv = _dinv(d0_ref, d1_ref)
    h = (a0_ref[...] + a1_ref[...] + g_ref[...]) * dinv + bias_ref[...]
    h = jnp.maximum(h, 0.0)
    o_ref[...] = jnp.dot(h, w_ref[...], preferred_element_type=_f32) * dinv


def _kc_body(a0_ref, a1_ref, g_ref, d0_ref, d1_ref, bias_ref, bb_ref,
             sums_ref, cnts_ref):
    i = pl.program_id(0)
    dinv = _dinv(d0_ref, d1_ref)
    h = (a0_ref[...] + a1_ref[...] + g_ref[...]) * dinv + bias_ref[...]
    h = jnp.maximum(h, 0.0)
    ids = bb_ref[:, 0:1].astype(jnp.int32)
    iota = lax.broadcasted_iota(jnp.int32, (ROW_BLK, N_GRAPHS), 1)
    oh = jnp.where(ids == iota, 1.0, 0.0)
    dn = (((0,), (0,)), ((), ()))
    ps = lax.dot_general(oh, h, dn, preferred_element_type=_f32)
    pc = lax.dot_general(oh, jnp.ones((ROW_BLK, D), _f32), dn,
                         preferred_element_type=_f32)

    @pl.when(i == 0)
    def _():
        sums_ref[...] = jnp.zeros_like(sums_ref)
        cnts_ref[...] = jnp.zeros_like(cnts_ref)

    sums_ref[...] += ps
    cnts_ref[...] += pc


def _kd_body(s_ref, c_ref, w_ref, bl_ref, o_ref):
    pooled = s_ref[...] / jnp.maximum(c_ref[...], 1.0)
    o_ref[...] = jnp.dot(pooled, w_ref[...],
                         preferred_element_type=_f32) + bl_ref[...]


def _row_spec(cols):
    return pl.BlockSpec((ROW_BLK, cols), lambda i: (i, 0))


def _full_spec(shape):
    return pl.BlockSpec(shape, lambda i: tuple(0 for _ in shape))


_KA = pl.pallas_call(
    _ka_body,
    grid=(GRID,),
    in_specs=[_row_spec(D), _row_spec(DEGW), _row_spec(DEGW),
              _full_spec((D, D))],
    out_specs=_row_spec(D),
    out_shape=jax.ShapeDtypeStruct((N, D), _f32),
)

_KB = pl.pallas_call(
    _kb_body,
    grid=(GRID,),
    in_specs=[_row_spec(D), _row_spec(D), _row_spec(D),
              _row_spec(DEGW), _row_spec(DEGW),
              _full_spec((1, D)), _full_spec((D, D))],
    out_specs=_row_spec(D),
    out_shape=jax.ShapeDtypeStruct((N, D), _f32),
)

_KC = pl.pallas_call(
    _kc_body,
    grid=(GRID,),
    in_specs=[_row_spec(D), _row_spec(D), _row_spec(D),
              _row_spec(DEGW), _row_spec(DEGW),
              _full_spec((1, D)), _row_spec(8)],
    out_specs=[_full_spec((N_GRAPHS, D)), _full_spec((N_GRAPHS, D))],
    out_shape=[jax.ShapeDtypeStruct((N_GRAPHS, D), _f32),
               jax.ShapeDtypeStruct((N_GRAPHS, D), _f32)],
)

_KD = pl.pallas_call(
    _kd_body,
    in_specs=[pl.BlockSpec((N_GRAPHS, D), lambda: (0, 0)),
              pl.BlockSpec((N_GRAPHS, D), lambda: (0, 0)),
              pl.BlockSpec((D, N_CLASSES), lambda: (0, 0)),
              pl.BlockSpec((1, N_CLASSES), lambda: (0, 0))],
    out_specs=pl.BlockSpec((N_GRAPHS, N_CLASSES), lambda: (0, 0)),
    out_shape=jax.ShapeDtypeStruct((N_GRAPHS, N_CLASSES), _f32),
)


def kernel(x, e, b, W1, b1, W2, b2, Wlin, blin):
    src = e[0].astype(jnp.int32)
    dst = e[1].astype(jnp.int32)
    pad = E_PAD - E
    pad_src = jnp.arange(pad, dtype=jnp.int32) % N
    pad_dst = N + jnp.arange(pad, dtype=jnp.int32) % (N_PAD - N)
    src_p = jnp.concatenate([src, pad_src]).reshape(NW, NCHUNKS, CHUNK)
    dst_f = jnp.concatenate([dst, pad_dst])

    deg_k, conv_k = _get_sc_kernels()
    degp = deg_k(dst_f)
    d0 = degp[0, :N]
    d1 = degp[1, :N]

    g1 = _KA(x, d0, d1, W1)
    accA = conv_k(src_p, dst_f, g1)
    g2 = _KB(accA[0, :N], accA[1, :N], g1, d0, d1,
             b1.reshape(1, D), W2)
    accB = conv_k(src_p, dst_f, g2)

    bb = jnp.broadcast_to(b.astype(_f32)[:, None], (N, 8))
    sums, cnts = _KC(accB[0, :N], accB[1, :N], g2, d0, d1,
                     b2.reshape(1, D), bb)
    return _KD(sums, cnts, Wlin, blin.reshape(1, N_CLASSES))


# speedup vs baseline: 3.4393x; 1.0740x over previous
import jax
import jax.numpy as jnp
from jax import lax
from jax.experimental import pallas as pl
from jax.experimental.pallas import tpu as pltpu
from jax.experimental.pallas import tpu_sc as plsc

N = 10000
E = 320000
D = 128
N_CLASSES = 10
N_GRAPHS = 64

NC = 2
NS = 16
NW = NC * NS

CHUNK = 128
NCHUNKS = 80
EDGES_PER_TILE = CHUNK * NCHUNKS
E_PAD = EDGES_PER_TILE * NW
N_PAD = 10240
ROWS_PER_TILE = N_PAD // NS
DEGW = 16

ROW_BLK = 1000
GRID = N // ROW_BLK

_f32 = jnp.float32


ZROWS = 16


def _zero_vmem_block(zbuf, ncols):
    def zrow(r, carry):
        for cc in range(ncols // 16):
            zbuf[r, pl.ds(cc * 16, 16)] = jnp.zeros((16,), _f32)
        return carry
    lax.fori_loop(0, ZROWS, zrow, 0)


def _zero_spmem_rows(acc, zbuf, row0, nrows):
    def zchunk(j, carry):
        pltpu.sync_copy(zbuf, acc.at[pl.ds(row0 + j * ZROWS, ZROWS)])
        return carry
    lax.fori_loop(0, nrows // ZROWS, zchunk, 0)


def _deg_body(dst_hbm, out_hbm, acc, zbuf, ones_v, dbuf0, dbuf1,
              semd0, semd1):
    c = lax.axis_index("c")
    s = lax.axis_index("s")
    wid = c * NS + s
    row0 = s * ROWS_PER_TILE
    _zero_vmem_block(zbuf, DEGW)
    _zero_spmem_rows(acc, zbuf, row0, ROWS_PER_TILE)

    def orow(r, carry):
        ones_v[r] = jnp.ones((16,), _f32)
        return carry
    lax.fori_loop(0, CHUNK, orow, 0)
    plsc.subcore_barrier()

    ebase = wid * EDGES_PER_TILE
    pltpu.async_copy(dst_hbm.at[pl.ds(ebase, CHUNK)], dbuf0, semd0)
    pltpu.async_copy(dst_hbm.at[pl.ds(ebase + CHUNK, CHUNK)], dbuf1, semd1)

    def half(i, dbuf, semd):
        pltpu.make_async_copy(dst_hbm.at[pl.ds(ebase, CHUNK)], dbuf,
                              semd).wait()
        pltpu.sync_copy(ones_v, acc.at[dbuf], add=True)

        @pl.when(i < NCHUNKS - 2)
        def _():
            pltpu.async_copy(
                dst_hbm.at[pl.ds(ebase + (i + 2) * CHUNK, CHUNK)], dbuf, semd)

    def step(j, carry):
        half(2 * j, dbuf0, semd0)
        half(2 * j + 1, dbuf1, semd1)
        return carry
    lax.fori_loop(0, NCHUNKS // 2, step, 0)
    plsc.subcore_barrier()
    pltpu.sync_copy(acc.at[pl.ds(row0, ROWS_PER_TILE)],
                    out_hbm.at[c, pl.ds(row0, ROWS_PER_TILE)])


def _conv_body(src_hbm, dst_hbm, g_hbm, out_hbm, acc, zbuf, idx_s,
               dbuf0, dbuf1, rows0, rows1, semg0, semg1, semd0, semd1):
    c = lax.axis_index("c")
    s = lax.axis_index("s")
    wid = c * NS + s
    row0 = s * ROWS_PER_TILE
    _zero_vmem_block(zbuf, D)
    _zero_spmem_rows(acc, zbuf, row0, ROWS_PER_TILE)
    pltpu.sync_copy(src_hbm.at[wid], idx_s)
    plsc.subcore_barrier()

    ebase = wid * EDGES_PER_TILE

    pltpu.async_copy(dst_hbm.at[pl.ds(ebase, CHUNK)], dbuf0, semd0)
    pltpu.async_copy(dst_hbm.at[pl.ds(ebase + CHUNK, CHUNK)], dbuf1, semd1)
    pltpu.async_copy(g_hbm.at[idx_s.at[0]], rows0, semg0)
    pltpu.async_copy(g_hbm.at[idx_s.at[1]], rows1, semg1)

    def half(i, dbuf, rows, semd, semg):
        pltpu.make_async_copy(g_hbm.at[idx_s.at[i]], rows, semg).wait()
        pltpu.make_async_copy(dst_hbm.at[pl.ds(ebase, CHUNK)], dbuf,
                              semd).wait()
        pltpu.sync_copy(rows, acc.at[dbuf], add=True)

        @pl.when(i < NCHUNKS - 2)
        def _():
            pltpu.async_copy(
                dst_hbm.at[pl.ds(ebase + (i + 2) * CHUNK, CHUNK)], dbuf, semd)
            pltpu.async_copy(g_hbm.at[idx_s.at[i + 2]], rows, semg)

    def body(j, carry):
        half(2 * j, dbuf0, rows0, semd0, semg0)
        half(2 * j + 1, dbuf1, rows1, semd1, semg1)
        return carry
    lax.fori_loop(0, NCHUNKS // 2, body, 0)
    plsc.subcore_barrier()
    pltpu.sync_copy(acc.at[pl.ds(row0, ROWS_PER_TILE)],
                    out_hbm.at[c, pl.ds(row0, ROWS_PER_TILE)])


_sc_kernels_cache = {}


def _get_sc_kernels():
    if "k" not in _sc_kernels_cache:
        mesh = plsc.VectorSubcoreMesh(core_axis_name="c", subcore_axis_name="s")
        deg_k = pl.kernel(
            _deg_body,
            out_type=jax.ShapeDtypeStruct((NC, N_PAD, DEGW), _f32),
            mesh=mesh,
            scratch_types=[
                pltpu.VMEM_SHARED((N_PAD, DEGW), _f32),
                pltpu.VMEM((ZROWS, DEGW), _f32),
                pltpu.VMEM((CHUNK, DEGW), _f32),
                pltpu.VMEM((CHUNK,), jnp.int32),
                pltpu.VMEM((CHUNK,), jnp.int32),
                pltpu.SemaphoreType.DMA,
                pltpu.SemaphoreType.DMA,
            ],
        )
        conv_k = pl.kernel(
            _conv_body,
            out_type=jax.ShapeDtypeStruct((NC, N_PAD, D), _f32),
            mesh=mesh,
            scratch_types=[
                pltpu.VMEM_SHARED((N_PAD, D), _f32),
                pltpu.VMEM((ZROWS, D), _f32),
                pltpu.VMEM((NCHUNKS, CHUNK), jnp.int32),
                pltpu.VMEM((CHUNK,), jnp.int32),
                pltpu.VMEM((CHUNK,), jnp.int32),
                pltpu.VMEM((CHUNK, D), _f32),
                pltpu.VMEM((CHUNK, D), _f32),
                pltpu.SemaphoreType.DMA,
                pltpu.SemaphoreType.DMA,
                pltpu.SemaphoreType.DMA,
                pltpu.SemaphoreType.DMA,
            ],
        )
        _sc_kernels_cache["k"] = (deg_k, conv_k)
    return _sc_kernels_cache["k"]


def _dinv(d0_ref, d1_ref):
    d = d0_ref[:, 0:1] + d1_ref[:, 0:1] + 1.0
    return lax.rsqrt(d)


def _kmm_body(x_ref, w_ref, o_ref):
    o_ref[...] = jnp.dot(x_ref[...], w_ref[...], preferred_element_type=_f32)


def _kscale_body(h_ref, d0_ref, d1_ref, o_ref):
    o_ref[...] = h_ref[...] * _dinv(d0_ref, d1_ref)


def _kb_body(a0_ref, a1_ref, g_ref, d0_ref, d1_ref, bias_ref, w_ref, o_ref):
    dinv = _dinv(d0_ref, d1_ref)
    h = (a0_ref[...] + a1_ref[...] + g_ref[...]) * dinv + bias_ref[...]
    h = jnp.maximum(h, 0.0)
    o_ref[...] = jnp.dot(h, w_ref[...], preferred_element_type=_f32) * dinv


def _kc_body(a0_ref, a1_ref, g_ref, d0_ref, d1_ref, bias_ref, bb_ref,
             sums_ref, cnts_ref):
    i = pl.program_id(0)
    dinv = _dinv(d0_ref, d1_ref)
    h = (a0_ref[...] + a1_ref[...] + g_ref[...]) * dinv + bias_ref[...]
    h = jnp.maximum(h, 0.0)
    ids = bb_ref[:, 0:1].astype(jnp.int32)
    iota = lax.broadcasted_iota(jnp.int32, (ROW_BLK, N_GRAPHS), 1)
    oh = jnp.where(ids == iota, 1.0, 0.0)
    dn = (((0,), (0,)), ((), ()))
    ps = lax.dot_general(oh, h, dn, preferred_element_type=_f32)
    pc = lax.dot_general(oh, jnp.ones((ROW_BLK, D), _f32), dn,
                         preferred_element_type=_f32)

    @pl.when(i == 0)
    def _():
        sums_ref[...] = jnp.zeros_like(sums_ref)
        cnts_ref[...] = jnp.zeros_like(cnts_ref)

    sums_ref[...] += ps
    cnts_ref[...] += pc


def _kd_body(s_ref, c_ref, w_ref, bl_ref, o_ref):
    pooled = s_ref[...] / jnp.maximum(c_ref[...], 1.0)
    o_ref[...] = jnp.dot(pooled, w_ref[...],
                         preferred_element_type=_f32) + bl_ref[...]


def _row_spec(cols):
    return pl.BlockSpec((ROW_BLK, cols), lambda i: (i, 0))


def _full_spec(shape):
    return pl.BlockSpec(shape, lambda i: tuple(0 for _ in shape))


_KMM = pl.pallas_call(
    _kmm_body,
    grid=(GRID,),
    in_specs=[_row_spec(D), _full_spec((D, D))],
    out_specs=_row_spec(D),
    out_shape=jax.ShapeDtypeStruct((N, D), _f32),
)

_KSCALE = pl.pallas_call(
    _kscale_body,
    grid=(GRID,),
    in_specs=[_row_spec(D), _row_spec(DEGW), _row_spec(DEGW)],
    out_specs=_row_spec(D),
    out_shape=jax.ShapeDtypeStruct((N, D), _f32),
)

_KB = pl.pallas_call(
    _kb_body,
    grid=(GRID,),
    in_specs=[_row_spec(D), _row_spec(D), _row_spec(D),
              _row_spec(DEGW), _row_spec(DEGW),
              _full_spec((1, D)), _full_spec((D, D))],
    out_specs=_row_spec(D),
    out_shape=jax.ShapeDtypeStruct((N, D), _f32),
)

_KC = pl.pallas_call(
    _kc_body,
    grid=(GRID,),
    in_specs=[_row_spec(D), _row_spec(D), _row_spec(D),
              _row_spec(DEGW), _row_spec(DEGW),
              _full_spec((1, D)), _row_spec(8)],
    out_specs=[_full_spec((N_GRAPHS, D)), _full_spec((N_GRAPHS, D))],
    out_shape=[jax.ShapeDtypeStruct((N_GRAPHS, D), _f32),
               jax.ShapeDtypeStruct((N_GRAPHS, D), _f32)],
)

_KD = pl.pallas_call(
    _kd_body,
    in_specs=[pl.BlockSpec((N_GRAPHS, D), lambda: (0, 0)),
              pl.BlockSpec((N_GRAPHS, D), lambda: (0, 0)),
              pl.BlockSpec((D, N_CLASSES), lambda: (0, 0)),
              pl.BlockSpec((1, N_CLASSES), lambda: (0, 0))],
    out_specs=pl.BlockSpec((N_GRAPHS, N_CLASSES), lambda: (0, 0)),
    out_shape=jax.ShapeDtypeStruct((N_GRAPHS, N_CLASSES), _f32),
)


def kernel(x, e, b, W1, b1, W2, b2, Wlin, blin):
    src = e[0].astype(jnp.int32)
    dst = e[1].astype(jnp.int32)
    pad = E_PAD - E
    pad_src = jnp.arange(pad, dtype=jnp.int32) % N
    pad_dst = N + jnp.arange(pad, dtype=jnp.int32) % (N_PAD - N)
    src_p = jnp.concatenate([src, pad_src]).reshape(NW, NCHUNKS, CHUNK)
    dst_f = jnp.concatenate([dst, pad_dst])

    deg_k, conv_k = _get_sc_kernels()
    mm1 = _KMM(x, W1)
    degp = deg_k(dst_f)
    d0 = degp[0, :N]
    d1 = degp[1, :N]

    g1 = _KSCALE(mm1, d0, d1)
    accA = conv_k(src_p, dst_f, g1)
    g2 = _KB(accA[0, :N], accA[1, :N], g1, d0, d1,
             b1.reshape(1, D), W2)
    accB = conv_k(src_p, dst_f, g2)

    bb = jnp.broadcast_to(b.astype(_f32)[:, None], (N, 8))
    sums, cnts = _KC(accB[0, :N], accB[1, :N], g2, d0, d1,
                     b2.reshape(1, D), bb)
    return _KD(sums, cnts, Wlin, blin.reshape(1, N_CLASSES))


# speedup vs baseline: 3.4426x; 1.0010x over previous
import jax
import jax.numpy as jnp
from jax import lax
from jax.experimental import pallas as pl
from jax.experimental.pallas import tpu as pltpu
from jax.experimental.pallas import tpu_sc as plsc

N = 10000
E = 320000
D = 128
N_CLASSES = 10
N_GRAPHS = 64

NC = 2
NS = 16
NW = NC * NS

CHUNK = 128
NCHUNKS = 80
EDGES_PER_TILE = CHUNK * NCHUNKS
E_PAD = EDGES_PER_TILE * NW
N_PAD = 10240
ROWS_PER_TILE = N_PAD // NS
DEGW = 16

ROW_BLK = 1000
GRID = N // ROW_BLK

_f32 = jnp.float32


ZROWS = 16


def _zero_vmem_block(zbuf, ncols):
    def zrow(r, carry):
        for cc in range(ncols // 16):
            zbuf[r, pl.ds(cc * 16, 16)] = jnp.zeros((16,), _f32)
        return carry
    lax.fori_loop(0, ZROWS, zrow, 0)


def _zero_spmem_rows(acc, zbuf, row0, nrows):
    def zchunk(j, carry):
        pltpu.sync_copy(zbuf, acc.at[pl.ds(row0 + j * ZROWS, ZROWS)])
        return carry
    lax.fori_loop(0, nrows // ZROWS, zchunk, 0)


def _deg_body(dst_hbm, out_hbm, acc, zbuf, ones_v, dbuf0, dbuf1,
              semd0, semd1):
    c = lax.axis_index("c")
    s = lax.axis_index("s")
    wid = c * NS + s
    row0 = s * ROWS_PER_TILE
    _zero_vmem_block(zbuf, DEGW)
    _zero_spmem_rows(acc, zbuf, row0, ROWS_PER_TILE)

    def orow(r, carry):
        ones_v[r] = jnp.ones((16,), _f32)
        return carry
    lax.fori_loop(0, CHUNK, orow, 0)
    plsc.subcore_barrier()

    ebase = wid * EDGES_PER_TILE
    pltpu.async_copy(dst_hbm.at[pl.ds(ebase, CHUNK)], dbuf0, semd0)
    pltpu.async_copy(dst_hbm.at[pl.ds(ebase + CHUNK, CHUNK)], dbuf1, semd1)

    def half(i, dbuf, semd):
        pltpu.make_async_copy(dst_hbm.at[pl.ds(ebase, CHUNK)], dbuf,
                              semd).wait()
        pltpu.sync_copy(ones_v, acc.at[dbuf], add=True)

        @pl.when(i < NCHUNKS - 2)
        def _():
            pltpu.async_copy(
                dst_hbm.at[pl.ds(ebase + (i + 2) * CHUNK, CHUNK)], dbuf, semd)

    def step(j, carry):
        half(2 * j, dbuf0, semd0)
        half(2 * j + 1, dbuf1, semd1)
        return carry
    lax.fori_loop(0, NCHUNKS // 2, step, 0)
    plsc.subcore_barrier()
    pltpu.sync_copy(acc.at[pl.ds(row0, ROWS_PER_TILE)],
                    out_hbm.at[c, pl.ds(row0, ROWS_PER_TILE)])


def _conv_body(src_hbm, dst_hbm, g_hbm, out_hbm, acc, zbuf, idx_s,
               dbuf0, dbuf1, rows0, rows1, semg0, semg1, semd0, semd1):
    c = lax.axis_index("c")
    s = lax.axis_index("s")
    wid = c * NS + s
    row0 = s * ROWS_PER_TILE
    _zero_vmem_block(zbuf, D)
    _zero_spmem_rows(acc, zbuf, row0, ROWS_PER_TILE)
    pltpu.sync_copy(src_hbm.at[wid], idx_s)
    plsc.subcore_barrier()

    ebase = wid * EDGES_PER_TILE

    pltpu.async_copy(dst_hbm.at[pl.ds(ebase, CHUNK)], dbuf0, semd0)
    pltpu.async_copy(dst_hbm.at[pl.ds(ebase + CHUNK, CHUNK)], dbuf1, semd1)
    pltpu.async_copy(g_hbm.at[idx_s.at[0]], rows0, semg0)
    pltpu.async_copy(g_hbm.at[idx_s.at[1]], rows1, semg1)

    def half(i, dbuf, rows, semd, semg):
        pltpu.make_async_copy(g_hbm.at[idx_s.at[i]], rows, semg).wait()
        pltpu.make_async_copy(dst_hbm.at[pl.ds(ebase, CHUNK)], dbuf,
                              semd).wait()
        pltpu.sync_copy(rows, acc.at[dbuf], add=True)

        @pl.when(i < NCHUNKS - 2)
        def _():
            pltpu.async_copy(
                dst_hbm.at[pl.ds(ebase + (i + 2) * CHUNK, CHUNK)], dbuf, semd)
            pltpu.async_copy(g_hbm.at[idx_s.at[i + 2]], rows, semg)

    def body(j, carry):
        half(2 * j, dbuf0, rows0, semd0, semg0)
        half(2 * j + 1, dbuf1, rows1, semd1, semg1)
        return carry
    lax.fori_loop(0, NCHUNKS // 2, body, 0)
    plsc.subcore_barrier()
    pltpu.sync_copy(acc.at[pl.ds(row0, ROWS_PER_TILE)],
                    out_hbm.at[c, pl.ds(row0, ROWS_PER_TILE)])


_sc_kernels_cache = {}


def _get_sc_kernels():
    if "k" not in _sc_kernels_cache:
        mesh = plsc.VectorSubcoreMesh(core_axis_name="c", subcore_axis_name="s")
        deg_k = pl.kernel(
            _deg_body,
            out_type=jax.ShapeDtypeStruct((NC, N_PAD, DEGW), _f32),
            mesh=mesh,
            scratch_types=[
                pltpu.VMEM_SHARED((N_PAD, DEGW), _f32),
                pltpu.VMEM((ZROWS, DEGW), _f32),
                pltpu.VMEM((CHUNK, DEGW), _f32),
                pltpu.VMEM((CHUNK,), jnp.int32),
                pltpu.VMEM((CHUNK,), jnp.int32),
                pltpu.SemaphoreType.DMA,
                pltpu.SemaphoreType.DMA,
            ],
        )
        conv_k = pl.kernel(
            _conv_body,
            out_type=jax.ShapeDtypeStruct((NC, N_PAD, D), _f32),
            mesh=mesh,
            scratch_types=[
                pltpu.VMEM_SHARED((N_PAD, D), _f32),
                pltpu.VMEM((ZROWS, D), _f32),
                pltpu.VMEM((NCHUNKS, CHUNK), jnp.int32),
                pltpu.VMEM((CHUNK,), jnp.int32),
                pltpu.VMEM((CHUNK,), jnp.int32),
                pltpu.VMEM((CHUNK, D), _f32),
                pltpu.VMEM((CHUNK, D), _f32),
                pltpu.SemaphoreType.DMA,
                pltpu.SemaphoreType.DMA,
                pltpu.SemaphoreType.DMA,
                pltpu.SemaphoreType.DMA,
            ],
        )
        _sc_kernels_cache["k"] = (deg_k, conv_k)
    return _sc_kernels_cache["k"]


def _dinv(d0_ref, d1_ref):
    d = d0_ref[:, 0:1] + d1_ref[:, 0:1] + 1.0
    return lax.rsqrt(d)


def _kmm_body(x_ref, w_ref, o_ref):
    o_ref[...] = jnp.dot(x_ref[...], w_ref[...], preferred_element_type=_f32)


def _kscale_body(h_ref, d0_ref, d1_ref, o_ref):
    o_ref[...] = h_ref[...] * _dinv(d0_ref, d1_ref)


def _kb_body(a0_ref, a1_ref, g_ref, d0_ref, d1_ref, bias_ref, w_ref, o_ref):
    dinv = _dinv(d0_ref, d1_ref)
    h = (a0_ref[...] + a1_ref[...] + g_ref[...]) * dinv + bias_ref[...]
    h = jnp.maximum(h, 0.0)
    o_ref[...] = jnp.dot(h, w_ref[...], preferred_element_type=_f32) * dinv


def _kc_body(a0_ref, a1_ref, g_ref, d0_ref, d1_ref, bias_ref, bb_ref,
             w_ref, bl_ref, o_ref, sums_ref, cnts_ref):
    i = pl.program_id(0)
    dinv = _dinv(d0_ref, d1_ref)
    h = (a0_ref[...] + a1_ref[...] + g_ref[...]) * dinv + bias_ref[...]
    h = jnp.maximum(h, 0.0)
    ids = bb_ref[:, 0:1].astype(jnp.int32)
    iota = lax.broadcasted_iota(jnp.int32, (ROW_BLK, N_GRAPHS), 1)
    oh = jnp.where(ids == iota, 1.0, 0.0)
    dn = (((0,), (0,)), ((), ()))
    ps = lax.dot_general(oh, h, dn, preferred_element_type=_f32)
    pc = lax.dot_general(oh, jnp.ones((ROW_BLK, D), _f32), dn,
                         preferred_element_type=_f32)

    @pl.when(i == 0)
    def _():
        sums_ref[...] = jnp.zeros_like(sums_ref)
        cnts_ref[...] = jnp.zeros_like(cnts_ref)

    sums_ref[...] += ps
    cnts_ref[...] += pc

    @pl.when(i == GRID - 1)
    def _():
        pooled = sums_ref[...] / jnp.maximum(cnts_ref[...], 1.0)
        o_ref[...] = jnp.dot(pooled, w_ref[...],
                             preferred_element_type=_f32) + bl_ref[...]


def _row_spec(cols):
    return pl.BlockSpec((ROW_BLK, cols), lambda i: (i, 0))


def _full_spec(shape):
    return pl.BlockSpec(shape, lambda i: tuple(0 for _ in shape))


_KMM = pl.pallas_call(
    _kmm_body,
    grid=(GRID,),
    in_specs=[_row_spec(D), _full_spec((D, D))],
    out_specs=_row_spec(D),
    out_shape=jax.ShapeDtypeStruct((N, D), _f32),
)

_KSCALE = pl.pallas_call(
    _kscale_body,
    grid=(GRID,),
    in_specs=[_row_spec(D), _row_spec(DEGW), _row_spec(DEGW)],
    out_specs=_row_spec(D),
    out_shape=jax.ShapeDtypeStruct((N, D), _f32),
)

_KB = pl.pallas_call(
    _kb_body,
    grid=(GRID,),
    in_specs=[_row_spec(D), _row_spec(D), _row_spec(D),
              _row_spec(DEGW), _row_spec(DEGW),
              _full_spec((1, D)), _full_spec((D, D))],
    out_specs=_row_spec(D),
    out_shape=jax.ShapeDtypeStruct((N, D), _f32),
)

_KC = pl.pallas_call(
    _kc_body,
    grid=(GRID,),
    in_specs=[_row_spec(D), _row_spec(D), _row_spec(D),
              _row_spec(DEGW), _row_spec(DEGW),
              _full_spec((1, D)), _row_spec(8),
              _full_spec((D, N_CLASSES)), _full_spec((1, N_CLASSES))],
    out_specs=_full_spec((N_GRAPHS, N_CLASSES)),
    out_shape=jax.ShapeDtypeStruct((N_GRAPHS, N_CLASSES), _f32),
    scratch_shapes=[pltpu.VMEM((N_GRAPHS, D), _f32),
                    pltpu.VMEM((N_GRAPHS, D), _f32)],
)


def kernel(x, e, b, W1, b1, W2, b2, Wlin, blin):
    src = e[0].astype(jnp.int32)
    dst = e[1].astype(jnp.int32)
    pad = E_PAD - E
    pad_src = jnp.arange(pad, dtype=jnp.int32) % N
    pad_dst = N + jnp.arange(pad, dtype=jnp.int32) % (N_PAD - N)
    src_p = jnp.concatenate([src, pad_src]).reshape(NW, NCHUNKS, CHUNK)
    dst_f = jnp.concatenate([dst, pad_dst])

    deg_k, conv_k = _get_sc_kernels()
    mm1 = _KMM(x, W1)
    degp = deg_k(dst_f)
    d0 = degp[0, :N]
    d1 = degp[1, :N]

    g1 = _KSCALE(mm1, d0, d1)
    accA = conv_k(src_p, dst_f, g1)
    g2 = _KB(accA[0, :N], accA[1, :N], g1, d0, d1,
             b1.reshape(1, D), W2)
    accB = conv_k(src_p, dst_f, g2)

    bb = jnp.broadcast_to(b.astype(_f32)[:, None], (N, 8))
    return _KC(accB[0, :N], accB[1, :N], g2, d0, d1,
               b2.reshape(1, D), bb, Wlin, blin.reshape(1, N_CLASSES))


# speedup vs baseline: 3.6655x; 1.0647x over previous
import jax
import jax.numpy as jnp
from jax import lax
from jax.experimental import pallas as pl
from jax.experimental.pallas import tpu as pltpu
from jax.experimental.pallas import tpu_sc as plsc

N = 10000
E = 320000
D = 128
N_CLASSES = 10
N_GRAPHS = 64

NC = 2
NS = 16
NW = NC * NS

CHUNK = 128
NCHUNKS = 80
EDGES_PER_TILE = CHUNK * NCHUNKS
E_PAD = EDGES_PER_TILE * NW
N_PAD = 10240
ROWS_PER_TILE = N_PAD // NS
DEGW = 16

ROW_BLK = 1000
GRID = N // ROW_BLK

_f32 = jnp.float32


ZROWS = 16


def _zero_vmem_block(zbuf, ncols):
    def zrow(r, carry):
        for cc in range(ncols // 16):
            zbuf[r, pl.ds(cc * 16, 16)] = jnp.zeros((16,), _f32)
        return carry
    lax.fori_loop(0, ZROWS, zrow, 0)


def _zero_spmem_rows(acc, zbuf, row0, nrows):
    def zchunk(j, carry):
        pltpu.sync_copy(zbuf, acc.at[pl.ds(row0 + j * ZROWS, ZROWS)])
        return carry
    lax.fori_loop(0, nrows // ZROWS, zchunk, 0)


def _deg_body(dst_hbm, out_hbm, acc, zbuf, ones_v, dbuf0, dbuf1,
              semd0, semd1):
    c = lax.axis_index("c")
    s = lax.axis_index("s")
    wid = c * NS + s
    row0 = s * ROWS_PER_TILE
    _zero_vmem_block(zbuf, DEGW)
    _zero_spmem_rows(acc, zbuf, row0, ROWS_PER_TILE)

    def orow(r, carry):
        ones_v[r] = jnp.ones((16,), _f32)
        return carry
    lax.fori_loop(0, CHUNK, orow, 0)
    plsc.subcore_barrier()

    ebase = wid * EDGES_PER_TILE
    pltpu.async_copy(dst_hbm.at[pl.ds(ebase, CHUNK)], dbuf0, semd0)
    pltpu.async_copy(dst_hbm.at[pl.ds(ebase + CHUNK, CHUNK)], dbuf1, semd1)

    def half(i, dbuf, semd):
        pltpu.make_async_copy(dst_hbm.at[pl.ds(ebase, CHUNK)], dbuf,
                              semd).wait()
        pltpu.sync_copy(ones_v, acc.at[dbuf], add=True)

        @pl.when(i < NCHUNKS - 2)
        def _():
            pltpu.async_copy(
                dst_hbm.at[pl.ds(ebase + (i + 2) * CHUNK, CHUNK)], dbuf, semd)

    def step(j, carry):
        half(2 * j, dbuf0, semd0)
        half(2 * j + 1, dbuf1, semd1)
        return carry
    lax.fori_loop(0, NCHUNKS // 2, step, 0)
    plsc.subcore_barrier()
    pltpu.sync_copy(acc.at[pl.ds(row0, ROWS_PER_TILE)],
                    out_hbm.at[c, pl.ds(row0, ROWS_PER_TILE)])


def _conv_body(src_hbm, dst_hbm, g_hbm, out_hbm, acc, zbuf, idx_s,
               dbuf0, dbuf1, rows0, rows1, semg0, semg1, semd0, semd1):
    c = lax.axis_index("c")
    s = lax.axis_index("s")
    wid = c * NS + s
    row0 = s * ROWS_PER_TILE
    _zero_vmem_block(zbuf, D)
    _zero_spmem_rows(acc, zbuf, row0, ROWS_PER_TILE)
    pltpu.sync_copy(src_hbm.at[wid], idx_s)
    plsc.subcore_barrier()

    ebase = wid * EDGES_PER_TILE

    pltpu.async_copy(dst_hbm.at[pl.ds(ebase, CHUNK)], dbuf0, semd0)
    pltpu.async_copy(dst_hbm.at[pl.ds(ebase + CHUNK, CHUNK)], dbuf1, semd1)
    pltpu.async_copy(g_hbm.at[idx_s.at[0]], rows0, semg0)
    pltpu.async_copy(g_hbm.at[idx_s.at[1]], rows1, semg1)

    def half(i, dbuf, rows, semd, semg):
        pltpu.make_async_copy(g_hbm.at[idx_s.at[i]], rows, semg).wait()
        pltpu.make_async_copy(dst_hbm.at[pl.ds(ebase, CHUNK)], dbuf,
                              semd).wait()
        pltpu.sync_copy(rows, acc.at[dbuf], add=True)

        @pl.when(i < NCHUNKS - 2)
        def _():
            pltpu.async_copy(
                dst_hbm.at[pl.ds(ebase + (i + 2) * CHUNK, CHUNK)], dbuf, semd)
            pltpu.async_copy(g_hbm.at[idx_s.at[i + 2]], rows, semg)

    def body(j, carry):
        half(2 * j, dbuf0, rows0, semd0, semg0)
        half(2 * j + 1, dbuf1, rows1, semd1, semg1)
        return carry
    lax.fori_loop(0, NCHUNKS // 2, body, 0)
    plsc.subcore_barrier()
    pltpu.sync_copy(acc.at[pl.ds(row0, ROWS_PER_TILE)],
                    out_hbm.at[c, pl.ds(row0, ROWS_PER_TILE)])


_sc_kernels_cache = {}


def _get_sc_kernels():
    if "k" not in _sc_kernels_cache:
        mesh = plsc.VectorSubcoreMesh(core_axis_name="c", subcore_axis_name="s")
        deg_k = pl.kernel(
            _deg_body,
            out_type=jax.ShapeDtypeStruct((NC, N_PAD, DEGW), _f32),
            mesh=mesh,
            scratch_types=[
                pltpu.VMEM_SHARED((N_PAD, DEGW), _f32),
                pltpu.VMEM((ZROWS, DEGW), _f32),
                pltpu.VMEM((CHUNK, DEGW), _f32),
                pltpu.VMEM((CHUNK,), jnp.int32),
                pltpu.VMEM((CHUNK,), jnp.int32),
                pltpu.SemaphoreType.DMA,
                pltpu.SemaphoreType.DMA,
            ],
        )
        conv_k = pl.kernel(
            _conv_body,
            out_type=jax.ShapeDtypeStruct((NC, N_PAD, D), _f32),
            mesh=mesh,
            scratch_types=[
                pltpu.VMEM_SHARED((N_PAD, D), _f32),
                pltpu.VMEM((ZROWS, D), _f32),
                pltpu.VMEM((NCHUNKS, CHUNK), jnp.int32),
                pltpu.VMEM((CHUNK,), jnp.int32),
                pltpu.VMEM((CHUNK,), jnp.int32),
                pltpu.VMEM((CHUNK, D), _f32),
                pltpu.VMEM((CHUNK, D), _f32),
                pltpu.SemaphoreType.DMA,
                pltpu.SemaphoreType.DMA,
                pltpu.SemaphoreType.DMA,
                pltpu.SemaphoreType.DMA,
            ],
        )
        _sc_kernels_cache["k"] = (deg_k, conv_k)
    return _sc_kernels_cache["k"]


def _dinv(deg_ref):
    d = deg_ref[0, :, 0:1] + deg_ref[1, :, 0:1] + 1.0
    return lax.rsqrt(d)


def _kmm_body(x_ref, w_ref, o_ref):
    o_ref[...] = jnp.dot(x_ref[...], w_ref[...], preferred_element_type=_f32)


def _kscale_body(h_ref, deg_ref, o_ref):
    o_ref[...] = h_ref[...] * _dinv(deg_ref)


def _kb_body(acc_ref, g_ref, deg_ref, bias_ref, w_ref, o_ref):
    dinv = _dinv(deg_ref)
    h = (acc_ref[0] + acc_ref[1] + g_ref[...]) * dinv + bias_ref[...]
    h = jnp.maximum(h, 0.0)
    o_ref[...] = jnp.dot(h, w_ref[...], preferred_element_type=_f32) * dinv


def _kc_body(acc_ref, g_ref, deg_ref, bias_ref, bb_ref,
             w_ref, bl_ref, o_ref, sums_ref, cnts_ref):
    i = pl.program_id(0)
    dinv = _dinv(deg_ref)
    h = (acc_ref[0] + acc_ref[1] + g_ref[...]) * dinv + bias_ref[...]
    h = jnp.maximum(h, 0.0)
    ids = bb_ref[:, 0:1].astype(jnp.int32)
    iota = lax.broadcasted_iota(jnp.int32, (ROW_BLK, N_GRAPHS), 1)
    oh = jnp.where(ids == iota, 1.0, 0.0)
    dn = (((0,), (0,)), ((), ()))
    ps = lax.dot_general(oh, h, dn, preferred_element_type=_f32)
    pc = lax.dot_general(oh, jnp.ones((ROW_BLK, D), _f32), dn,
                         preferred_element_type=_f32)

    @pl.when(i == 0)
    def _():
        sums_ref[...] = jnp.zeros_like(sums_ref)
        cnts_ref[...] = jnp.zeros_like(cnts_ref)

    sums_ref[...] += ps
    cnts_ref[...] += pc

    @pl.when(i == GRID - 1)
    def _():
        pooled = sums_ref[...] / jnp.maximum(cnts_ref[...], 1.0)
        o_ref[...] = jnp.dot(pooled, w_ref[...],
                             preferred_element_type=_f32) + bl_ref[...]


def _row_spec(cols):
    return pl.BlockSpec((ROW_BLK, cols), lambda i: (i, 0))


def _acc_spec(cols):
    return pl.BlockSpec((2, ROW_BLK, cols), lambda i: (0, i, 0))


def _full_spec(shape):
    return pl.BlockSpec(shape, lambda i: tuple(0 for _ in shape))


_KMM = pl.pallas_call(
    _kmm_body,
    grid=(GRID,),
    in_specs=[_row_spec(D), _full_spec((D, D))],
    out_specs=_row_spec(D),
    out_shape=jax.ShapeDtypeStruct((N, D), _f32),
)

_KSCALE = pl.pallas_call(
    _kscale_body,
    grid=(GRID,),
    in_specs=[_row_spec(D), _acc_spec(DEGW)],
    out_specs=_row_spec(D),
    out_shape=jax.ShapeDtypeStruct((N, D), _f32),
)

_KB = pl.pallas_call(
    _kb_body,
    grid=(GRID,),
    in_specs=[_acc_spec(D), _row_spec(D), _acc_spec(DEGW),
              _full_spec((1, D)), _full_spec((D, D))],
    out_specs=_row_spec(D),
    out_shape=jax.ShapeDtypeStruct((N, D), _f32),
)

_KC = pl.pallas_call(
    _kc_body,
    grid=(GRID,),
    in_specs=[_acc_spec(D), _row_spec(D), _acc_spec(DEGW),
              _full_spec((1, D)), _row_spec(8),
              _full_spec((D, N_CLASSES)), _full_spec((1, N_CLASSES))],
    out_specs=_full_spec((N_GRAPHS, N_CLASSES)),
    out_shape=jax.ShapeDtypeStruct((N_GRAPHS, N_CLASSES), _f32),
    scratch_shapes=[pltpu.VMEM((N_GRAPHS, D), _f32),
                    pltpu.VMEM((N_GRAPHS, D), _f32)],
)


def kernel(x, e, b, W1, b1, W2, b2, Wlin, blin):
    src = e[0].astype(jnp.int32)
    dst = e[1].astype(jnp.int32)
    pad = E_PAD - E
    pad_src = jnp.arange(pad, dtype=jnp.int32) % N
    pad_dst = N + jnp.arange(pad, dtype=jnp.int32) % (N_PAD - N)
    src_p = jnp.concatenate([src, pad_src]).reshape(NW, NCHUNKS, CHUNK)
    dst_f = jnp.concatenate([dst, pad_dst])

    deg_k, conv_k = _get_sc_kernels()
    mm1 = _KMM(x, W1)
    degp = deg_k(dst_f)

    g1 = _KSCALE(mm1, degp)
    accA = conv_k(src_p, dst_f, g1)
    g2 = _KB(accA, g1, degp, b1.reshape(1, D), W2)
    accB = conv_k(src_p, dst_f, g2)

    bb = jnp.broadcast_to(b.astype(_f32)[:, None], (N, 8))
    return _KC(accB, g2, degp,
               b2.reshape(1, D), bb, Wlin, blin.reshape(1, N_CLASSES))


# speedup vs baseline: 3.6737x; 1.0022x over previous
import jax
import jax.numpy as jnp
from jax import lax
from jax.experimental import pallas as pl
from jax.experimental.pallas import tpu as pltpu
from jax.experimental.pallas import tpu_sc as plsc

N = 10000
E = 320000
D = 128
N_CLASSES = 10
N_GRAPHS = 64

NC = 2
NS = 16
NW = NC * NS

CHUNK = 128
NCHUNKS = 80
EDGES_PER_TILE = CHUNK * NCHUNKS
E_PAD = EDGES_PER_TILE * NW
N_PAD = 10240
ROWS_PER_TILE = N_PAD // NS
DEGW = 16

ROW_BLK = 1000
GRID = N // ROW_BLK

_f32 = jnp.float32


ZROWS = 16


def _zero_vmem_block(zbuf, ncols):
    def zrow(r, carry):
        for cc in range(ncols // 16):
            zbuf[r, pl.ds(cc * 16, 16)] = jnp.zeros((16,), _f32)
        return carry
    lax.fori_loop(0, ZROWS, zrow, 0)


def _zero_spmem_rows(acc, zbuf, row0, nrows):
    def zchunk(j, carry):
        pltpu.sync_copy(zbuf, acc.at[pl.ds(row0 + j * ZROWS, ZROWS)])
        return carry
    lax.fori_loop(0, nrows // ZROWS, zchunk, 0)


def _deg_body(dst_hbm, out_hbm, acc, zbuf, ones_v, dbuf0, dbuf1,
              semd0, semd1):
    c = lax.axis_index("c")
    s = lax.axis_index("s")
    wid = c * NS + s
    row0 = s * ROWS_PER_TILE
    _zero_vmem_block(zbuf, DEGW)
    _zero_spmem_rows(acc, zbuf, row0, ROWS_PER_TILE)

    def orow(r, carry):
        ones_v[r] = jnp.ones((16,), _f32)
        return carry
    lax.fori_loop(0, CHUNK, orow, 0)
    plsc.subcore_barrier()

    ebase = wid * EDGES_PER_TILE
    pltpu.async_copy(dst_hbm.at[pl.ds(ebase, CHUNK)], dbuf0, semd0)
    pltpu.async_copy(dst_hbm.at[pl.ds(ebase + CHUNK, CHUNK)], dbuf1, semd1)

    def half(i, dbuf, semd):
        pltpu.make_async_copy(dst_hbm.at[pl.ds(ebase, CHUNK)], dbuf,
                              semd).wait()
        pltpu.sync_copy(ones_v, acc.at[dbuf], add=True)

        @pl.when(i < NCHUNKS - 2)
        def _():
            pltpu.async_copy(
                dst_hbm.at[pl.ds(ebase + (i + 2) * CHUNK, CHUNK)], dbuf, semd)

    def step(j, carry):
        half(2 * j, dbuf0, semd0)
        half(2 * j + 1, dbuf1, semd1)
        return carry
    lax.fori_loop(0, NCHUNKS // 2, step, 0)
    plsc.subcore_barrier()
    pltpu.sync_copy(acc.at[pl.ds(row0, ROWS_PER_TILE)],
                    out_hbm.at[c, pl.ds(row0, ROWS_PER_TILE)])


def _conv_body(src_hbm, dst_hbm, g_hbm, out_hbm, acc, zbuf, idx_s,
               dbuf0, dbuf1, rows0, rows1, semg0, semg1, semd0, semd1):
    c = lax.axis_index("c")
    s = lax.axis_index("s")
    wid = c * NS + s
    row0 = s * ROWS_PER_TILE
    _zero_vmem_block(zbuf, D)
    _zero_spmem_rows(acc, zbuf, row0, ROWS_PER_TILE)
    pltpu.sync_copy(src_hbm.at[wid], idx_s)
    plsc.subcore_barrier()

    ebase = wid * EDGES_PER_TILE

    pltpu.async_copy(dst_hbm.at[pl.ds(ebase, CHUNK)], dbuf0, semd0)
    pltpu.async_copy(dst_hbm.at[pl.ds(ebase + CHUNK, CHUNK)], dbuf1, semd1)
    pltpu.async_copy(g_hbm.at[idx_s.at[0]], rows0, semg0)
    pltpu.async_copy(g_hbm.at[idx_s.at[1]], rows1, semg1)

    def half(i, dbuf, rows, semd, semg):
        pltpu.make_async_copy(g_hbm.at[idx_s.at[i]], rows, semg).wait()
        pltpu.make_async_copy(dst_hbm.at[pl.ds(ebase, CHUNK)], dbuf,
                              semd).wait()
        pltpu.sync_copy(rows, acc.at[dbuf], add=True)

        @pl.when(i < NCHUNKS - 2)
        def _():
            pltpu.async_copy(
                dst_hbm.at[pl.ds(ebase + (i + 2) * CHUNK, CHUNK)], dbuf, semd)
            pltpu.async_copy(g_hbm.at[idx_s.at[i + 2]], rows, semg)

    def body(j, carry):
        half(2 * j, dbuf0, rows0, semd0, semg0)
        half(2 * j + 1, dbuf1, rows1, semd1, semg1)
        return carry
    lax.fori_loop(0, NCHUNKS // 2, body, 0)
    plsc.subcore_barrier()
    pltpu.sync_copy(acc.at[pl.ds(row0, ROWS_PER_TILE)],
                    out_hbm.at[c, pl.ds(row0, ROWS_PER_TILE)])


_sc_kernels_cache = {}


def _get_sc_kernels():
    if "k" not in _sc_kernels_cache:
        mesh = plsc.VectorSubcoreMesh(core_axis_name="c", subcore_axis_name="s")
        deg_k = pl.kernel(
            _deg_body,
            out_type=jax.ShapeDtypeStruct((NC, N_PAD, DEGW), _f32),
            mesh=mesh,
            scratch_types=[
                pltpu.VMEM_SHARED((N_PAD, DEGW), _f32),
                pltpu.VMEM((ZROWS, DEGW), _f32),
                pltpu.VMEM((CHUNK, DEGW), _f32),
                pltpu.VMEM((CHUNK,), jnp.int32),
                pltpu.VMEM((CHUNK,), jnp.int32),
                pltpu.SemaphoreType.DMA,
                pltpu.SemaphoreType.DMA,
            ],
        )
        conv_k = pl.kernel(
            _conv_body,
            out_type=jax.ShapeDtypeStruct((NC, N_PAD, D), _f32),
            mesh=mesh,
            scratch_types=[
                pltpu.VMEM_SHARED((N_PAD, D), _f32),
                pltpu.VMEM((ZROWS, D), _f32),
                pltpu.VMEM((NCHUNKS, CHUNK), jnp.int32),
                pltpu.VMEM((CHUNK,), jnp.int32),
                pltpu.VMEM((CHUNK,), jnp.int32),
                pltpu.VMEM((CHUNK, D), _f32),
                pltpu.VMEM((CHUNK, D), _f32),
                pltpu.SemaphoreType.DMA,
                pltpu.SemaphoreType.DMA,
                pltpu.SemaphoreType.DMA,
                pltpu.SemaphoreType.DMA,
            ],
        )
        _sc_kernels_cache["k"] = (deg_k, conv_k)
    return _sc_kernels_cache["k"]


def _dinv(deg_ref):
    d = deg_ref[0, :, 0:1] + deg_ref[1, :, 0:1] + 1.0
    return lax.rsqrt(d)


def _ka_body(x_ref, deg_ref, w_ref, o_ref):
    o_ref[...] = jnp.dot(x_ref[...], w_ref[...],
                         preferred_element_type=_f32) * _dinv(deg_ref)


def _kb_body(acc_ref, g_ref, deg_ref, bias_ref, w_ref, o_ref):
    dinv = _dinv(deg_ref)
    h = (acc_ref[0] + acc_ref[1] + g_ref[...]) * dinv + bias_ref[...]
    h = jnp.maximum(h, 0.0)
    o_ref[...] = jnp.dot(h, w_ref[...], preferred_element_type=_f32) * dinv


def _kc_body(acc_ref, g_ref, deg_ref, bias_ref, bb_ref,
             w_ref, bl_ref, o_ref, sums_ref, cnts_ref):
    i = pl.program_id(0)
    dinv = _dinv(deg_ref)
    h = (acc_ref[0] + acc_ref[1] + g_ref[...]) * dinv + bias_ref[...]
    h = jnp.maximum(h, 0.0)
    ids = bb_ref[:, 0:1].astype(jnp.int32)
    iota = lax.broadcasted_iota(jnp.int32, (ROW_BLK, N_GRAPHS), 1)
    oh = jnp.where(ids == iota, 1.0, 0.0)
    dn = (((0,), (0,)), ((), ()))
    ps = lax.dot_general(oh, h, dn, preferred_element_type=_f32)
    pc = lax.dot_general(oh, jnp.ones((ROW_BLK, D), _f32), dn,
                         preferred_element_type=_f32)

    @pl.when(i == 0)
    def _():
        sums_ref[...] = jnp.zeros_like(sums_ref)
        cnts_ref[...] = jnp.zeros_like(cnts_ref)

    sums_ref[...] += ps
    cnts_ref[...] += pc

    @pl.when(i == GRID - 1)
    def _():
        pooled = sums_ref[...] / jnp.maximum(cnts_ref[...], 1.0)
        o_ref[...] = jnp.dot(pooled, w_ref[...],
                             preferred_element_type=_f32) + bl_ref[...]


def _row_spec(cols):
    return pl.BlockSpec((ROW_BLK, cols), lambda i: (i, 0))


def _acc_spec(cols):
    return pl.BlockSpec((2, ROW_BLK, cols), lambda i: (0, i, 0))


def _full_spec(shape):
    return pl.BlockSpec(shape, lambda i: tuple(0 for _ in shape))


_KA = pl.pallas_call(
    _ka_body,
    grid=(GRID,),
    in_specs=[_row_spec(D), _acc_spec(DEGW), _full_spec((D, D))],
    out_specs=_row_spec(D),
    out_shape=jax.ShapeDtypeStruct((N, D), _f32),
)

_KB = pl.pallas_call(
    _kb_body,
    grid=(GRID,),
    in_specs=[_acc_spec(D), _row_spec(D), _acc_spec(DEGW),
              _full_spec((1, D)), _full_spec((D, D))],
    out_specs=_row_spec(D),
    out_shape=jax.ShapeDtypeStruct((N, D), _f32),
)

_KC = pl.pallas_call(
    _kc_body,
    grid=(GRID,),
    in_specs=[_acc_spec(D), _row_spec(D), _acc_spec(DEGW),
              _full_spec((1, D)), _row_spec(8),
              _full_spec((D, N_CLASSES)), _full_spec((1, N_CLASSES))],
    out_specs=_full_spec((N_GRAPHS, N_CLASSES)),
    out_shape=jax.ShapeDtypeStruct((N_GRAPHS, N_CLASSES), _f32),
    scratch_shapes=[pltpu.VMEM((N_GRAPHS, D), _f32),
                    pltpu.VMEM((N_GRAPHS, D), _f32)],
)


def kernel(x, e, b, W1, b1, W2, b2, Wlin, blin):
    src = e[0].astype(jnp.int32)
    dst = e[1].astype(jnp.int32)
    pad = E_PAD - E
    pad_src = jnp.arange(pad, dtype=jnp.int32) % N
    pad_dst = N + jnp.arange(pad, dtype=jnp.int32) % (N_PAD - N)
    src_p = jnp.concatenate([src, pad_src]).reshape(NW, NCHUNKS, CHUNK)
    dst_f = jnp.concatenate([dst, pad_dst])

    deg_k, conv_k = _get_sc_kernels()
    degp = deg_k(dst_f)

    g1 = _KA(x, degp, W1)
    accA = conv_k(src_p, dst_f, g1)
    g2 = _KB(accA, g1, degp, b1.reshape(1, D), W2)
    accB = conv_k(src_p, dst_f, g2)

    bb = jnp.broadcast_to(b.astype(_f32)[:, None], (N, 8))
    return _KC(accB, g2, degp,
               b2.reshape(1, D), bb, Wlin, blin.reshape(1, N_CLASSES))
